# baseline (speedup 1.0000x reference)
import jax
import jax.numpy as jnp
from jax import lax
from jax.experimental import pallas as pl
from jax.experimental.pallas import tpu as pltpu

_MAX_MASK_LEN = 20
_TT = 512


def _mask_params(B, D):
    key = jax.random.key(42)
    k1, k2 = jax.random.split(key)
    hi = min(_MAX_MASK_LEN, D // 4)
    mask_len = jax.random.randint(k1, (1,), 1, hi)
    ml = mask_len[0]
    mask_start = jax.random.randint(k2, (B,), 0, jnp.maximum(1, D - ml))
    return ml, mask_start


def _body(s_ref, x_ref, o_ref):
    b = pl.program_id(0)
    ml = s_ref[0]
    start = s_ref[1 + b]
    col = lax.broadcasted_iota(jnp.int32, (1, 1, x_ref.shape[-1]), 2)
    mask = (col >= start) & (col < start + ml)
    o_ref[...] = jnp.where(mask, jnp.float32(0.0), x_ref[...])


def kernel(mean):
    B, T, D = mean.shape
    ml, mask_start = _mask_params(B, D)
    scalars = jnp.concatenate([ml[None], mask_start]).astype(jnp.int32)

    grid_spec = pltpu.PrefetchScalarGridSpec(
        num_scalar_prefetch=1,
        grid=(B, T // _TT),
        in_specs=[pl.BlockSpec((1, _TT, D), lambda b, t, s: (b, t, 0))],
        out_specs=pl.BlockSpec((1, _TT, D), lambda b, t, s: (b, t, 0)),
    )
    return pl.pallas_call(
        _body,
        grid_spec=grid_spec,
        out_shape=jax.ShapeDtypeStruct((B, T, D), mean.dtype),
    )(scalars, mean)


# speedup vs baseline: 1.2088x; 1.2088x over previous
import jax
import jax.numpy as jnp
from jax import lax
from jax.experimental import pallas as pl
from jax.experimental.pallas import tpu as pltpu

_MAX_MASK_LEN = 20
_TT = 1024


def _mask_params(B, D):
    key = jax.random.key(42)
    k1, k2 = jax.random.split(key)
    hi = min(_MAX_MASK_LEN, D // 4)
    mask_len = jax.random.randint(k1, (1,), 1, hi)
    ml = mask_len[0]
    mask_start = jax.random.randint(k2, (B,), 0, jnp.maximum(1, D - ml))
    return ml, mask_start


def _body(s_ref, x_ref, o_ref):
    b = pl.program_id(0)
    ml = s_ref[0]
    start = s_ref[1 + b]
    col = lax.broadcasted_iota(jnp.int32, (1, 1, x_ref.shape[-1]), 2)
    mask = (col >= start) & (col < start + ml)
    o_ref[...] = jnp.where(mask, jnp.float32(0.0), x_ref[...])


def kernel(mean):
    B, T, D = mean.shape
    ml, mask_start = _mask_params(B, D)
    scalars = jnp.concatenate([ml[None], mask_start]).astype(jnp.int32)

    grid_spec = pltpu.PrefetchScalarGridSpec(
        num_scalar_prefetch=1,
        grid=(B, T // _TT),
        in_specs=[pl.BlockSpec((1, _TT, D), lambda b, t, s: (b, t, 0))],
        out_specs=pl.BlockSpec((1, _TT, D), lambda b, t, s: (b, t, 0)),
    )
    return pl.pallas_call(
        _body,
        grid_spec=grid_spec,
        out_shape=jax.ShapeDtypeStruct((B, T, D), mean.dtype),
    )(scalars, mean)


# speedup vs baseline: 1.2674x; 1.0485x over previous
import jax
import jax.numpy as jnp
from jax import lax
from jax.experimental import pallas as pl
from jax.experimental.pallas import tpu as pltpu

_MAX_MASK_LEN = 20
_TT = 2048


def _mask_params(B, D):
    key = jax.random.key(42)
    k1, k2 = jax.random.split(key)
    hi = min(_MAX_MASK_LEN, D // 4)
    mask_len = jax.random.randint(k1, (1,), 1, hi)
    ml = mask_len[0]
    mask_start = jax.random.randint(k2, (B,), 0, jnp.maximum(1, D - ml))
    return ml, mask_start


def _body(s_ref, x_ref, o_ref):
    b = pl.program_id(0)
    ml = s_ref[0]
    start = s_ref[1 + b]
    col = lax.broadcasted_iota(jnp.int32, (1, 1, x_ref.shape[-1]), 2)
    mask = (col >= start) & (col < start + ml)
    o_ref[...] = jnp.where(mask, jnp.float32(0.0), x_ref[...])


def kernel(mean):
    B, T, D = mean.shape
    ml, mask_start = _mask_params(B, D)
    scalars = jnp.concatenate([ml[None], mask_start]).astype(jnp.int32)

    grid_spec = pltpu.PrefetchScalarGridSpec(
        num_scalar_prefetch=1,
        grid=(B, T // _TT),
        in_specs=[pl.BlockSpec((1, _TT, D), lambda b, t, s: (b, t, 0))],
        out_specs=pl.BlockSpec((1, _TT, D), lambda b, t, s: (b, t, 0)),
    )
    return pl.pallas_call(
        _body,
        grid_spec=grid_spec,
        out_shape=jax.ShapeDtypeStruct((B, T, D), mean.dtype),
    )(scalars, mean)
